# Initial kernel scaffold; baseline (speedup 1.0000x reference)
import jax
import jax.numpy as jnp
from jax.experimental import pallas as pl


def kernel(values, weight, row_idx, col_idx):
    raise NotImplementedError("write your pallas kernel here")


# speedup vs baseline: 8.9115x; 8.9115x over previous
import functools
import math

import jax
import jax.numpy as jnp
from jax import lax
from jax.experimental import pallas as pl
from jax.experimental.pallas import tpu as pltpu
from jax.experimental.pallas import tpu_sc as plsc

N = 16384
D = 64
L = 16
NW = 32
RPW = N // NW
C = 1024
GB = 128


def _make_kernel(nnz_pad: int):
    nb = nnz_pad // C
    iters = max(1, math.ceil(math.log2(nb)))
    mesh = plsc.VectorSubcoreMesh(core_axis_name="c", subcore_axis_name="s")

    @functools.partial(
        pl.kernel,
        mesh=mesh,
        out_type=jax.ShapeDtypeStruct((N, D), jnp.float32),
        compiler_params=pltpu.CompilerParams(use_tc_tiling_on_sc=False),
        scratch_types=[
            pltpu.VMEM((L,), jnp.int32),
            pltpu.VMEM((C,), jnp.int32),
            pltpu.VMEM((C,), jnp.int32),
            pltpu.VMEM((C,), jnp.float32),
            pltpu.VMEM((C, D), jnp.float32),
            pltpu.VMEM((RPW + 1, D), jnp.float32),
            pltpu.SemaphoreType.DMA,
        ],
    )
    def spmm(values_hbm, weight_hbm, row_hbm, col_hbm, out_hbm,
             probe_v, col_v, row_v, val_v, rows_v, acc_v, gsem):
        wid = lax.axis_index("s") * 2 + lax.axis_index("c")
        base = wid * RPW

        def first_chunk_ge(target):
            def body(_, carry):
                lo, hi = carry
                mid = jnp.minimum((lo + hi) // 2, nb - 1)
                pltpu.sync_copy(row_hbm.at[pl.ds(mid * C, L)], probe_v)
                ge = probe_v[pl.ds(0, L)][0] >= target
                return (jnp.where(ge, lo, mid + 1), jnp.where(ge, mid, hi))
            lo, hi = lax.fori_loop(0, iters, body, (jnp.int32(0), jnp.int32(nb)))
            return hi

        j_lo = first_chunk_ge(base)
        j_hi = first_chunk_ge(base + RPW)
        j_start = jnp.maximum(j_lo - 1, 0)

        zero = jnp.zeros((L,), jnp.float32)
        def zbody(r, _):
            for dcol in range(D // L):
                acc_v[r, pl.ds(dcol * L, L)] = zero
            return 0
        lax.fori_loop(0, RPW + 1, zbody, 0)

        def chunk_body(j, _):
            off = j * C
            pltpu.sync_copy(col_hbm.at[pl.ds(off, C)], col_v)
            pltpu.sync_copy(row_hbm.at[pl.ds(off, C)], row_v)
            pltpu.sync_copy(values_hbm.at[pl.ds(off, C)], val_v)
            copies = []
            for b in range(C // GB):
                copies.append(pltpu.async_copy(
                    weight_hbm.at[col_v.at[pl.ds(b * GB, GB)]],
                    rows_v.at[pl.ds(b * GB, GB)], gsem))
            for cp in copies:
                cp.wait()

            def grp_body(g, _):
                rl = row_v[pl.ds(g * L, L)] - base
                ok = (rl >= 0) & (rl < RPW)
                rr = jnp.where(ok, rl, RPW)
                vm = jnp.where(ok, val_v[pl.ds(g * L, L)], 0.0)
                for j in range(L):
                    r_j = rr[j]
                    v_j = vm[j]
                    i = g * L + j
                    for dcol in range(D // L):
                        sl = pl.ds(dcol * L, L)
                        acc_v[r_j, sl] = acc_v[r_j, sl] + v_j * rows_v[i, sl]
                return 0
            lax.fori_loop(0, C // L, grp_body, 0)
            return 0

        lax.fori_loop(j_start, j_hi, chunk_body, 0)

        pltpu.sync_copy(acc_v.at[pl.ds(0, RPW)], out_hbm.at[pl.ds(base, RPW)])

    return spmm


def kernel(values, weight, row_idx, col_idx):
    nnz = values.shape[0]
    nnz_pad = (nnz // C + 2) * C
    pad = nnz_pad - nnz
    values_p = jnp.pad(values, (0, pad))
    row_p = jnp.pad(row_idx.astype(jnp.int32), (0, pad), constant_values=N)
    col_p = jnp.pad(col_idx.astype(jnp.int32), (0, pad))
    return _make_kernel(nnz_pad)(values_p, weight, row_p, col_p)


# speedup vs baseline: 22.5036x; 2.5252x over previous
import functools
import math

import jax
import jax.numpy as jnp
from jax import lax
from jax.experimental import pallas as pl
from jax.experimental.pallas import tpu as pltpu
from jax.experimental.pallas import tpu_sc as plsc

N = 16384
D = 64
L = 16
NW = 32
RPW = N // NW
C = 1024
GB = 128


def _make_kernel(nnz_pad: int):
    nb = nnz_pad // C
    iters = max(1, math.ceil(math.log2(nb)))
    mesh = plsc.VectorSubcoreMesh(core_axis_name="c", subcore_axis_name="s")

    @functools.partial(
        pl.kernel,
        mesh=mesh,
        out_type=jax.ShapeDtypeStruct((N, D), jnp.float32),
        compiler_params=pltpu.CompilerParams(use_tc_tiling_on_sc=False),
        scratch_types=[
            pltpu.VMEM((L,), jnp.int32),
            pltpu.VMEM((C,), jnp.int32),
            pltpu.VMEM((C,), jnp.int32),
            pltpu.VMEM((C,), jnp.float32),
            pltpu.VMEM((C, D), jnp.float32),
            pltpu.VMEM((RPW + 1, D), jnp.float32),
            pltpu.SemaphoreType.DMA,
        ],
    )
    def spmm(values_hbm, weight_hbm, row_hbm, col_hbm, out_hbm,
             probe_v, col_v, row_v, val_v, rows_v, acc_v, gsem):
        wid = lax.axis_index("s") * 2 + lax.axis_index("c")
        base = wid * RPW

        def first_chunk_ge(target):
            def body(_, carry):
                lo, hi = carry
                mid = jnp.minimum((lo + hi) // 2, nb - 1)
                pltpu.sync_copy(row_hbm.at[pl.ds(mid * C, L)], probe_v)
                ge = probe_v[pl.ds(0, L)][0] >= target
                return (jnp.where(ge, lo, mid + 1), jnp.where(ge, mid, hi))
            lo, hi = lax.fori_loop(0, iters, body, (jnp.int32(0), jnp.int32(nb)))
            return hi

        j_lo = first_chunk_ge(base)
        j_hi = first_chunk_ge(base + RPW)
        j_start = jnp.maximum(j_lo - 1, 0)

        zero = jnp.zeros((L,), jnp.float32)
        def zbody(r, _):
            for dcol in range(D // L):
                acc_v[r, pl.ds(dcol * L, L)] = zero
            return 0
        lax.fori_loop(0, RPW + 1, zbody, 0)

        def chunk_body(j, _):
            off = j * C
            pltpu.sync_copy(col_hbm.at[pl.ds(off, C)], col_v)
            pltpu.sync_copy(row_hbm.at[pl.ds(off, C)], row_v)
            pltpu.sync_copy(values_hbm.at[pl.ds(off, C)], val_v)
            copies = []
            for b in range(C // GB):
                copies.append(pltpu.async_copy(
                    weight_hbm.at[col_v.at[pl.ds(b * GB, GB)]],
                    rows_v.at[pl.ds(b * GB, GB)], gsem))
            for cp in copies:
                cp.wait()

            def grp_body(g, _):
                gl = g * L
                rl = row_v[pl.ds(gl, L)] - base
                ok = (rl >= 0) & (rl < RPW)
                rr = jnp.where(ok, rl, RPW)
                vm = jnp.where(ok, val_v[pl.ds(gl, L)], 0.0)
                uniform = rl[0] == rl[L - 1]

                def fast(_):
                    r0 = rr[0]
                    nd = D // L
                    p = [jnp.zeros((L,), jnp.float32) for _ in range(nd)]
                    q = [jnp.zeros((L,), jnp.float32) for _ in range(nd)]
                    for j in range(L):
                        v_j = vm[j]
                        tgt = p if j % 2 == 0 else q
                        for d in range(nd):
                            tgt[d] = tgt[d] + v_j * rows_v[gl + j, pl.ds(d * L, L)]
                    for d in range(nd):
                        sl = pl.ds(d * L, L)
                        acc_v[r0, sl] = acc_v[r0, sl] + (p[d] + q[d])
                    return 0

                def slow(_):
                    for j in range(L):
                        r_j = rr[j]
                        v_j = vm[j]
                        for d in range(D // L):
                            sl = pl.ds(d * L, L)
                            acc_v[r_j, sl] = acc_v[r_j, sl] + v_j * rows_v[gl + j, sl]
                    return 0

                lax.cond(uniform, fast, slow, 0)
                return 0
            lax.fori_loop(0, C // L, grp_body, 0)
            return 0

        lax.fori_loop(j_start, j_hi, chunk_body, 0)

        pltpu.sync_copy(acc_v.at[pl.ds(0, RPW)], out_hbm.at[pl.ds(base, RPW)])

    return spmm


def kernel(values, weight, row_idx, col_idx):
    nnz = values.shape[0]
    nnz_pad = (nnz // C + 2) * C
    pad = nnz_pad - nnz
    values_p = jnp.pad(values, (0, pad))
    row_p = jnp.pad(row_idx.astype(jnp.int32), (0, pad), constant_values=N)
    col_p = jnp.pad(col_idx.astype(jnp.int32), (0, pad))
    return _make_kernel(nnz_pad)(values_p, weight, row_p, col_p)


# speedup vs baseline: 31.9410x; 1.4194x over previous
import functools
import math

import jax
import jax.numpy as jnp
from jax import lax
from jax.experimental import pallas as pl
from jax.experimental.pallas import tpu as pltpu
from jax.experimental.pallas import tpu_sc as plsc

N = 16384
D = 64
L = 16
NW = 32
RPW = N // NW
C = 512
GB = 128


def _make_kernel(nnz_pad: int):
    nb = nnz_pad // C
    iters = max(1, math.ceil(math.log2(nb)))
    mesh = plsc.VectorSubcoreMesh(core_axis_name="c", subcore_axis_name="s")

    @functools.partial(
        pl.kernel,
        mesh=mesh,
        out_type=jax.ShapeDtypeStruct((N, D), jnp.float32),
        compiler_params=pltpu.CompilerParams(use_tc_tiling_on_sc=False),
        scratch_types=[
            pltpu.VMEM((L,), jnp.int32),
            pltpu.VMEM((C,), jnp.int32),
            pltpu.VMEM((C,), jnp.int32),
            pltpu.VMEM((C,), jnp.int32),
            pltpu.VMEM((C,), jnp.int32),
            pltpu.VMEM((C,), jnp.float32),
            pltpu.VMEM((C,), jnp.float32),
            pltpu.VMEM((C, D), jnp.float32),
            pltpu.VMEM((C, D), jnp.float32),
            pltpu.VMEM((RPW + 1, D), jnp.float32),
            pltpu.SemaphoreType.DMA,
            pltpu.SemaphoreType.DMA,
            pltpu.SemaphoreType.DMA,
            pltpu.SemaphoreType.DMA,
        ],
    )
    def spmm(values_hbm, weight_hbm, row_hbm, col_hbm, out_hbm,
             probe_v, col0, col1, row0, row1, val0, val1, rows0, rows1,
             acc_v, lsem0, lsem1, gsem0, gsem1):
        wid = lax.axis_index("s") * 2 + lax.axis_index("c")
        base = wid * RPW
        bufs = ((col0, row0, val0, rows0, lsem0, gsem0),
                (col1, row1, val1, rows1, lsem1, gsem1))

        def first_chunk_ge(target):
            def body(_, carry):
                lo, hi = carry
                mid = jnp.minimum((lo + hi) // 2, nb - 1)
                pltpu.sync_copy(row_hbm.at[pl.ds(mid * C, L)], probe_v)
                ge = probe_v[pl.ds(0, L)][0] >= target
                return (jnp.where(ge, lo, mid + 1), jnp.where(ge, mid, hi))
            lo, hi = lax.fori_loop(0, iters, body, (jnp.int32(0), jnp.int32(nb)))
            return hi

        j_lo = first_chunk_ge(base)
        j_hi = first_chunk_ge(base + RPW)
        j_start = jnp.maximum(j_lo - 1, 0)

        def linear_descs(j, b):
            col_v, row_v, val_v, _, lsem, _ = bufs[b]
            off = j * C
            return ((col_hbm.at[pl.ds(off, C)], col_v, lsem),
                    (row_hbm.at[pl.ds(off, C)], row_v, lsem),
                    (values_hbm.at[pl.ds(off, C)], val_v, lsem))

        def linear_start(j, b):
            for args in linear_descs(j, b):
                pltpu.async_copy(*args)

        def linear_wait(j, b):
            for args in linear_descs(j, b):
                pltpu.make_async_copy(*args).wait()

        def gather_descs(b):
            col_v, _, _, rows_v, _, gsem = bufs[b]
            return tuple(
                (weight_hbm.at[col_v.at[pl.ds(g * GB, GB)]],
                 rows_v.at[pl.ds(g * GB, GB)], gsem)
                for g in range(C // GB))

        def gather_start(b):
            for args in gather_descs(b):
                pltpu.async_copy(*args)

        def gather_wait(b):
            for args in gather_descs(b):
                pltpu.make_async_copy(*args).wait()

        def compute(b):
            _, row_v, val_v, rows_v, _, _ = bufs[b]

            def grp_body(g, _):
                gl = g * L
                rl = row_v[pl.ds(gl, L)] - base
                ok = (rl >= 0) & (rl < RPW)
                rr = jnp.where(ok, rl, RPW)
                vm = jnp.where(ok, val_v[pl.ds(gl, L)], 0.0)
                uniform = rl[0] == rl[L - 1]

                def fast(_):
                    r0 = rr[0]
                    nd = D // L
                    p = [jnp.zeros((L,), jnp.float32) for _ in range(nd)]
                    q = [jnp.zeros((L,), jnp.float32) for _ in range(nd)]
                    for j in range(L):
                        v_j = vm[j]
                        tgt = p if j % 2 == 0 else q
                        for d in range(nd):
                            tgt[d] = tgt[d] + v_j * rows_v[gl + j, pl.ds(d * L, L)]
                    for d in range(nd):
                        sl = pl.ds(d * L, L)
                        acc_v[r0, sl] = acc_v[r0, sl] + (p[d] + q[d])
                    return 0

                def slow(_):
                    for j in range(L):
                        r_j = rr[j]
                        v_j = vm[j]
                        for d in range(D // L):
                            sl = pl.ds(d * L, L)
                            acc_v[r_j, sl] = acc_v[r_j, sl] + v_j * rows_v[gl + j, sl]
                    return 0

                lax.cond(uniform, fast, slow, 0)
                return 0
            lax.fori_loop(0, C // L, grp_body, 0)

        linear_start(j_start, 0)

        zero = jnp.zeros((L,), jnp.float32)
        def zbody(r, _):
            for dcol in range(D // L):
                acc_v[r, pl.ds(dcol * L, L)] = zero
            return 0
        lax.fori_loop(0, RPW + 1, zbody, 0)

        linear_wait(j_start, 0)
        gather_start(0)
        linear_start(j_start + 1, 1)

        num = j_hi - j_start
        pairs = jnp.maximum(1, (num + 1) // 2)

        def pair_body(k, _):
            j0 = j_start + 2 * k
            linear_wait(j0 + 1, 1)
            gather_start(1)
            gather_wait(0)
            compute(0)
            linear_start(j0 + 2, 0)
            linear_wait(j0 + 2, 0)
            gather_start(0)
            gather_wait(1)
            compute(1)
            linear_start(j0 + 3, 1)
            return 0
        lax.fori_loop(0, pairs, pair_body, 0)

        gather_wait(0)
        linear_wait(j_start, 1)

        pltpu.sync_copy(acc_v.at[pl.ds(0, RPW)], out_hbm.at[pl.ds(base, RPW)])

    return spmm


def kernel(values, weight, row_idx, col_idx):
    nnz = values.shape[0]
    nnz_pad = (nnz // C + 6) * C
    pad = nnz_pad - nnz
    values_p = jnp.pad(values, (0, pad))
    row_p = jnp.pad(row_idx.astype(jnp.int32), (0, pad), constant_values=N)
    col_p = jnp.pad(col_idx.astype(jnp.int32), (0, pad))
    return _make_kernel(nnz_pad)(values_p, weight, row_p, col_p)
